# Initial kernel scaffold; baseline (speedup 1.0000x reference)
import jax
import jax.numpy as jnp
from jax.experimental import pallas as pl


def kernel(x, edge_index, batch, W1a, b1a, g1, be1, W1b, b1b, W2a, b2a, g2, be2, W2b, b2b, Wl1, bl1, Wl2, bl2):
    raise NotImplementedError("write your pallas kernel here")


# speedup vs baseline: 2.3937x; 2.3937x over previous
import functools

import jax
import jax.numpy as jnp
from jax import lax
from jax.experimental import pallas as pl
from jax.experimental.pallas import tpu as pltpu
from jax.experimental.pallas import tpu_sc as plsc

N = 10000
H = 128
B = 64
E = 320000
NC = 2
NS = 16
NW = NC * NS
CH = 128
NCHUNK = 80
EPAD = NW * NCHUNK * CH
NACC = 10240
ZCH = NACC // NS // CH
ROWS_T = 624
CPR = 104
TAIL0 = NS * ROWS_T
TAILR = N - TAIL0

def _sc_agg_body(x_hbm, src_hbm, dst_hbm, out_hbm, src_v, dst_v, rows_v, acc_sh, sem):
    c = lax.axis_index("c")
    s = lax.axis_index("s")
    wid = s * NC + c

    pltpu.sync_copy(src_hbm.at[wid], src_v)
    pltpu.sync_copy(dst_hbm.at[wid], dst_v)

    def _zrow(i, _):
        def _zcol(j, __):
            rows_v[i, pl.ds(j * 16, 16)] = jnp.zeros((16,), jnp.float32)
            return 0
        return lax.fori_loop(0, H // 16, _zcol, 0)
    lax.fori_loop(0, CH, _zrow, 0)
    for z in range(ZCH):
        pltpu.sync_copy(rows_v, acc_sh.at[pl.ds((s * ZCH + z) * CH, CH)])
    plsc.subcore_barrier()

    def _body(j, _):
        pltpu.async_copy(x_hbm.at[src_v.at[j]], rows_v, sem).wait()
        pltpu.sync_copy(rows_v, acc_sh.at[dst_v.at[j]], add=True)
        return 0
    lax.fori_loop(0, NCHUNK, _body, 0)
    plsc.subcore_barrier()

    for k in range(ROWS_T // CPR):
        r0 = s * ROWS_T + k * CPR
        pltpu.sync_copy(acc_sh.at[pl.ds(r0, CPR)], rows_v.at[pl.ds(0, CPR)])
        pltpu.sync_copy(rows_v.at[pl.ds(0, CPR)], out_hbm.at[c, pl.ds(r0, CPR)])

    @pl.when(s == 0)
    def _tail():
        pltpu.sync_copy(acc_sh.at[pl.ds(TAIL0, TAILR)], rows_v.at[pl.ds(0, TAILR)])
        pltpu.sync_copy(rows_v.at[pl.ds(0, TAILR)], out_hbm.at[c, pl.ds(TAIL0, TAILR)])


@functools.lru_cache(maxsize=None)
def _make_sc_agg():
    mesh = plsc.VectorSubcoreMesh(core_axis_name="c", subcore_axis_name="s")
    return pl.kernel(
        _sc_agg_body,
        mesh=mesh,
        out_type=jax.ShapeDtypeStruct((NC, N, H), jnp.float32),
        scratch_types=[
            pltpu.VMEM((NCHUNK, CH), jnp.int32),
            pltpu.VMEM((NCHUNK, CH), jnp.int32),
            pltpu.VMEM((CH, H), jnp.float32),
            pltpu.VMEM_SHARED((NACC, H), jnp.float32),
            pltpu.SemaphoreType.DMA,
        ],
    )


def _dense1_body(x_ref, agg_ref, Wa_ref, ba_ref, g_ref, be_ref, Wb_ref, bb_ref, out_ref):
    h = x_ref[...] + agg_ref[0] + agg_ref[1]
    h = jnp.dot(h, Wa_ref[...], preferred_element_type=jnp.float32) + ba_ref[...]
    m = jnp.mean(h, axis=0, keepdims=True)
    cc = h - m
    v = jnp.mean(cc * cc, axis=0, keepdims=True)
    h = g_ref[...] * cc * lax.rsqrt(v + 1e-5) + be_ref[...]
    h = jnp.maximum(h, 0.0)
    h = jnp.dot(h, Wb_ref[...], preferred_element_type=jnp.float32) + bb_ref[...]
    out_ref[...] = jnp.maximum(h, 0.0)


_dense1 = pl.pallas_call(
    _dense1_body,
    out_shape=jax.ShapeDtypeStruct((N, H), jnp.float32),
)


def _dense2_body(h1_ref, agg_ref, batch_ref, Wa_ref, ba_ref, g_ref, be_ref,
                 Wb_ref, bb_ref, Wl1_ref, bl1_ref, Wl2_ref, bl2_ref,
                 sig_ref, lin_ref):
    h1 = h1_ref[...]
    h = h1 + agg_ref[0] + agg_ref[1]
    h = jnp.dot(h, Wa_ref[...], preferred_element_type=jnp.float32) + ba_ref[...]
    m = jnp.mean(h, axis=0, keepdims=True)
    cc = h - m
    v = jnp.mean(cc * cc, axis=0, keepdims=True)
    h = g_ref[...] * cc * lax.rsqrt(v + 1e-5) + be_ref[...]
    h = jnp.maximum(h, 0.0)
    h = jnp.dot(h, Wb_ref[...], preferred_element_type=jnp.float32) + bb_ref[...]
    h2 = jnp.maximum(h, 0.0)

    bvec = batch_ref[...]
    seg = lax.broadcasted_iota(jnp.int32, (1, B), 1)
    onehot = (bvec == seg).astype(jnp.float32)
    dn = (((0,), (0,)), ((), ()))
    h1_sum = lax.dot_general(onehot, h1, dn, preferred_element_type=jnp.float32)
    h2_sum = lax.dot_general(onehot, h2, dn, preferred_element_type=jnp.float32)

    neg = jnp.float32(-jnp.inf)
    rowid = lax.broadcasted_iota(jnp.int32, (B, 1), 0)

    def _seg_max(b, carry):
        m1acc, m2acc = carry
        mask = bvec == b
        m1 = jnp.max(jnp.where(mask, h1, neg), axis=0, keepdims=True)
        m2 = jnp.max(jnp.where(mask, h2, neg), axis=0, keepdims=True)
        rowsel = rowid == b
        return (jnp.where(rowsel, m1, m1acc), jnp.where(rowsel, m2, m2acc))

    h1_max, h2_max = lax.fori_loop(
        0, B, _seg_max,
        (jnp.full((B, H), neg), jnp.full((B, H), neg)))

    hp = jnp.concatenate((h1_sum, h2_sum, h1_max, h2_max), axis=1)
    hh = jnp.dot(hp, Wl1_ref[...], preferred_element_type=jnp.float32) + bl1_ref[...]
    hh = jnp.maximum(hh, 0.0)
    hh = jnp.dot(hh, Wl2_ref[...], preferred_element_type=jnp.float32) + bl2_ref[...]
    lin_ref[...] = hh
    sig_ref[...] = jax.nn.sigmoid(hh)


_dense2 = pl.pallas_call(
    _dense2_body,
    out_shape=(jax.ShapeDtypeStruct((B, 1), jnp.float32),
               jax.ShapeDtypeStruct((B, 1), jnp.float32)),
)


def kernel(x, edge_index, batch, W1a, b1a, g1, be1, W1b, b1b, W2a, b2a, g2, be2,
           W2b, b2b, Wl1, bl1, Wl2, bl2):
    src = edge_index[0]
    dst = edge_index[1]
    pad = EPAD - E
    src3 = jnp.concatenate([src, jnp.zeros((pad,), jnp.int32)]).reshape(NW, NCHUNK, CH)
    dst3 = jnp.concatenate([dst, jnp.full((pad,), N, jnp.int32)]).reshape(NW, NCHUNK, CH)

    _sc_agg = _make_sc_agg()
    agg1 = _sc_agg(x, src3, dst3)
    h1 = _dense1(x, agg1, W1a, b1a.reshape(1, H), g1.reshape(1, H),
                 be1.reshape(1, H), W1b, b1b.reshape(1, H))
    agg2 = _sc_agg(h1, src3, dst3)
    return _dense2(h1, agg2, batch.reshape(N, 1), W2a, b2a.reshape(1, H),
                   g2.reshape(1, H), be2.reshape(1, H), W2b, b2b.reshape(1, H),
                   Wl1, bl1.reshape(1, 4 * H), Wl2, bl2.reshape(1, 1))


# speedup vs baseline: 2.6087x; 1.0898x over previous
import functools

import jax
import jax.numpy as jnp
from jax import lax
from jax.experimental import pallas as pl
from jax.experimental.pallas import tpu as pltpu
from jax.experimental.pallas import tpu_sc as plsc

N = 10000
H = 128
B = 64
E = 320000
NC = 2
NS = 16
NW = NC * NS
CH = 128
NCHUNK = 80
EPAD = NW * NCHUNK * CH
NB = 2
NACC = 10240
ZR = 80
ROWS_T = 624
CPR = 104
TAIL0 = NS * ROWS_T
TAILR = N - TAIL0

def _sc_agg_body(x_hbm, src_hbm, dst_hbm, out_hbm, src_v, d0, d1,
                 b0, b1, acc_sh, g0, g1, s0, s1, e0, e1):
    bufs = (b0, b1)
    dring = (d0, d1)
    gsem = (g0, g1)
    ssem = (s0, s1)
    dsem = (e0, e1)
    c = lax.axis_index("c")
    s = lax.axis_index("s")
    wid = s * NC + c

    pltpu.sync_copy(src_hbm.at[wid], src_v)

    rows_v = bufs[0]

    def _zrow(i, _):
        def _zcol(j, __):
            rows_v[i, pl.ds(j * 16, 16)] = jnp.zeros((16,), jnp.float32)
            return 0
        return lax.fori_loop(0, H // 16, _zcol, 0)
    lax.fori_loop(0, ZR, _zrow, 0)
    nz = NACC // NS // ZR
    for z in range(nz):
        pltpu.sync_copy(rows_v.at[pl.ds(0, ZR)],
                        acc_sh.at[pl.ds(s * (nz * ZR) + z * ZR, ZR)])
    plsc.subcore_barrier()

    def _gather(j, b):
        pltpu.async_copy(x_hbm.at[src_v.at[j]], bufs[b], gsem[b])

    for b in range(NB):
        pltpu.async_copy(dst_hbm.at[wid, b], dring[b], dsem[b])
        _gather(b, b)

    def _outer(t, _):
        j0 = t * NB
        for b in range(NB):
            pltpu.make_async_copy(x_hbm.at[src_v.at[j0 + b]], bufs[b], gsem[b]).wait()
            pltpu.make_async_copy(dst_hbm.at[wid, j0 + b], dring[b], dsem[b]).wait()
            pltpu.async_copy(bufs[b], acc_sh.at[dring[b]], ssem[b], add=True)
        for b in range(NB):
            pltpu.make_async_copy(bufs[b], acc_sh.at[dring[b]], ssem[b]).wait()

            @pl.when(j0 + NB + b < NCHUNK)
            def _():
                pltpu.async_copy(dst_hbm.at[wid, j0 + NB + b], dring[b], dsem[b])
                _gather(j0 + NB + b, b)
        return 0
    lax.fori_loop(0, NCHUNK // NB, _outer, 0)
    plsc.subcore_barrier()

    nk = ROWS_T // CPR
    for k in range(nk):
        bk = bufs[k % 2]
        r0 = s * ROWS_T + k * CPR
        if k >= 2:
            rp = s * ROWS_T + (k - 2) * CPR
            pltpu.make_async_copy(bk.at[pl.ds(0, CPR)],
                                  out_hbm.at[c, pl.ds(rp, CPR)], ssem[k % 2]).wait()
        pltpu.sync_copy(acc_sh.at[pl.ds(r0, CPR)], bk.at[pl.ds(0, CPR)])
        pltpu.async_copy(bk.at[pl.ds(0, CPR)], out_hbm.at[c, pl.ds(r0, CPR)], ssem[k % 2])
    for k in range(nk - 2, nk):
        bk = bufs[k % 2]
        r0 = s * ROWS_T + k * CPR
        pltpu.make_async_copy(bk.at[pl.ds(0, CPR)],
                              out_hbm.at[c, pl.ds(r0, CPR)], ssem[k % 2]).wait()

    @pl.when(s == 0)
    def _tail():
        pltpu.sync_copy(acc_sh.at[pl.ds(TAIL0, TAILR)], b0.at[pl.ds(0, TAILR)])
        pltpu.sync_copy(b0.at[pl.ds(0, TAILR)], out_hbm.at[c, pl.ds(TAIL0, TAILR)])


@functools.lru_cache(maxsize=None)
def _make_sc_agg():
    mesh = plsc.VectorSubcoreMesh(core_axis_name="c", subcore_axis_name="s")
    return pl.kernel(
        _sc_agg_body,
        mesh=mesh,
        out_type=jax.ShapeDtypeStruct((NC, N, H), jnp.float32),
        scratch_types=(
            [pltpu.VMEM((NCHUNK, CH), jnp.int32)]
            + [pltpu.VMEM((CH,), jnp.int32)] * NB
            + [pltpu.VMEM((CH, H), jnp.float32)] * NB
            + [pltpu.VMEM_SHARED((NACC, H), jnp.float32)]
            + [pltpu.SemaphoreType.DMA] * (3 * NB)
        ),
    )


def _dense1_body(x_ref, agg_ref, Wa_ref, ba_ref, g_ref, be_ref, Wb_ref, bb_ref, out_ref):
    h = x_ref[...] + agg_ref[0] + agg_ref[1]
    h = jnp.dot(h, Wa_ref[...], preferred_element_type=jnp.float32) + ba_ref[...]
    m = jnp.mean(h, axis=0, keepdims=True)
    cc = h - m
    v = jnp.mean(cc * cc, axis=0, keepdims=True)
    h = g_ref[...] * cc * lax.rsqrt(v + 1e-5) + be_ref[...]
    h = jnp.maximum(h, 0.0)
    h = jnp.dot(h, Wb_ref[...], preferred_element_type=jnp.float32) + bb_ref[...]
    out_ref[...] = jnp.maximum(h, 0.0)


_dense1 = pl.pallas_call(
    _dense1_body,
    out_shape=jax.ShapeDtypeStruct((N, H), jnp.float32),
)


def _dense2_body(h1_ref, agg_ref, batch_ref, Wa_ref, ba_ref, g_ref, be_ref,
                 Wb_ref, bb_ref, Wl1_ref, bl1_ref, Wl2_ref, bl2_ref,
                 sig_ref, lin_ref):
    h1 = h1_ref[...]
    h = h1 + agg_ref[0] + agg_ref[1]
    h = jnp.dot(h, Wa_ref[...], preferred_element_type=jnp.float32) + ba_ref[...]
    m = jnp.mean(h, axis=0, keepdims=True)
    cc = h - m
    v = jnp.mean(cc * cc, axis=0, keepdims=True)
    h = g_ref[...] * cc * lax.rsqrt(v + 1e-5) + be_ref[...]
    h = jnp.maximum(h, 0.0)
    h = jnp.dot(h, Wb_ref[...], preferred_element_type=jnp.float32) + bb_ref[...]
    h2 = jnp.maximum(h, 0.0)

    bvec = batch_ref[...]
    seg = lax.broadcasted_iota(jnp.int32, (1, B), 1)
    onehot = (bvec == seg).astype(jnp.float32)
    dn = (((0,), (0,)), ((), ()))
    h1_sum = lax.dot_general(onehot, h1, dn, preferred_element_type=jnp.float32)
    h2_sum = lax.dot_general(onehot, h2, dn, preferred_element_type=jnp.float32)

    neg = jnp.float32(-jnp.inf)
    rowid = lax.broadcasted_iota(jnp.int32, (B, 1), 0)

    def _seg_max(b, carry):
        m1acc, m2acc = carry
        mask = bvec == b
        m1 = jnp.max(jnp.where(mask, h1, neg), axis=0, keepdims=True)
        m2 = jnp.max(jnp.where(mask, h2, neg), axis=0, keepdims=True)
        rowsel = rowid == b
        return (jnp.where(rowsel, m1, m1acc), jnp.where(rowsel, m2, m2acc))

    h1_max, h2_max = lax.fori_loop(
        0, B, _seg_max,
        (jnp.full((B, H), neg), jnp.full((B, H), neg)))

    hp = jnp.concatenate((h1_sum, h2_sum, h1_max, h2_max), axis=1)
    hh = jnp.dot(hp, Wl1_ref[...], preferred_element_type=jnp.float32) + bl1_ref[...]
    hh = jnp.maximum(hh, 0.0)
    hh = jnp.dot(hh, Wl2_ref[...], preferred_element_type=jnp.float32) + bl2_ref[...]
    lin_ref[...] = hh
    sig_ref[...] = jax.nn.sigmoid(hh)


_dense2 = pl.pallas_call(
    _dense2_body,
    out_shape=(jax.ShapeDtypeStruct((B, 1), jnp.float32),
               jax.ShapeDtypeStruct((B, 1), jnp.float32)),
)


def kernel(x, edge_index, batch, W1a, b1a, g1, be1, W1b, b1b, W2a, b2a, g2, be2,
           W2b, b2b, Wl1, bl1, Wl2, bl2):
    src = edge_index[0]
    dst = edge_index[1]
    pad = EPAD - E
    src3 = jnp.concatenate([src, jnp.zeros((pad,), jnp.int32)]).reshape(NW, NCHUNK, CH)
    dst3 = jnp.concatenate([dst, jnp.full((pad,), N, jnp.int32)]).reshape(NW, NCHUNK, CH)

    _sc_agg = _make_sc_agg()
    agg1 = _sc_agg(x, src3, dst3)
    h1 = _dense1(x, agg1, W1a, b1a.reshape(1, H), g1.reshape(1, H),
                 be1.reshape(1, H), W1b, b1b.reshape(1, H))
    agg2 = _sc_agg(h1, src3, dst3)
    return _dense2(h1, agg2, batch.reshape(N, 1), W2a, b2a.reshape(1, H),
                   g2.reshape(1, H), be2.reshape(1, H), W2b, b2b.reshape(1, H),
                   Wl1, bl1.reshape(1, 4 * H), Wl2, bl2.reshape(1, 1))
